# baseline (speedup 1.0000x reference)
import functools

import jax
import jax.numpy as jnp
from jax import lax
from jax.experimental import pallas as pl
from jax.experimental.pallas import tpu as pltpu
from jax.experimental.pallas import tpu_sc as plsc

F = 26
V = 100000
D = 32
B = 16384

NC, NS, L = 2, 16, 16
NW = NC * NS
ROWS = B * F
RPW = ROWS // NW
CH = 512
NCH = RPW // CH


def _sc_embedding_gather(table2d, idx2d):
    mesh = plsc.VectorSubcoreMesh(core_axis_name="c", subcore_axis_name="s")

    @functools.partial(
        pl.kernel,
        mesh=mesh,
        out_type=jax.ShapeDtypeStruct((ROWS, D), jnp.float32),
        compiler_params=pltpu.CompilerParams(use_tc_tiling_on_sc=False),
        scratch_types=[
            pltpu.VMEM((RPW,), jnp.int32),
            pltpu.VMEM((CH, D), jnp.float32),
            pltpu.VMEM((CH, D), jnp.float32),
            pltpu.SemaphoreType.DMA,
            pltpu.SemaphoreType.DMA,
        ],
    )
    def k(table_hbm, idx_hbm, out_hbm, idx_v, buf0, buf1, sem0, sem1):
        wid = lax.axis_index("s") * NC + lax.axis_index("c")
        base = wid * RPW

        pltpu.sync_copy(idx_hbm.at[pl.ds(base, RPW)], idx_v)

        lanes = lax.iota(jnp.int32, 16)

        def add_offsets(t, _):
            p = base + t * L
            fld = (p + lanes) % F
            idx_v[pl.ds(t * L, L)] = idx_v[pl.ds(t * L, L)] + fld * V
            return 0

        lax.fori_loop(0, RPW // L, add_offsets, 0)

        bufs = (buf0, buf1)
        sems = (sem0, sem1)

        def gather(j, buf, sem):
            return pltpu.async_copy(
                table_hbm.at[idx_v.at[pl.ds(j * CH, CH)]], buf, sem)

        cp = gather(0, bufs[0], sems[0])
        for j in range(1, NCH):
            nxt = gather(j, bufs[j % 2], sems[j % 2])
            cp.wait()
            pltpu.sync_copy(bufs[(j - 1) % 2],
                            out_hbm.at[pl.ds(base + (j - 1) * CH, CH)])
            cp = nxt
        cp.wait()
        pltpu.sync_copy(bufs[(NCH - 1) % 2],
                        out_hbm.at[pl.ds(base + (NCH - 1) * CH, CH)])

    return k(table2d, idx2d)


def kernel(indices, tables):
    idx_flat = indices.astype(jnp.int32).reshape(ROWS)
    table2d = tables.reshape(F * V, D)
    out = _sc_embedding_gather(table2d, idx_flat)
    return out.reshape(B, F * D)


# speedup vs baseline: 3.4433x; 3.4433x over previous
import functools

import jax
import jax.numpy as jnp
from jax import lax
from jax.experimental import pallas as pl
from jax.experimental.pallas import tpu as pltpu
from jax.experimental.pallas import tpu_sc as plsc

F = 26
V = 100000
D = 32
B = 16384

NC, NS, L = 2, 16, 16
NW = NC * NS
BCH = 8192
NBCH = B // BCH
NVEC = BCH // L


def _sc_col_gather(ttab, idx_t):
    mesh = plsc.VectorSubcoreMesh(core_axis_name="c", subcore_axis_name="s")

    @functools.partial(
        pl.kernel,
        mesh=mesh,
        out_type=jax.ShapeDtypeStruct((F * D, B), jnp.float32),
        compiler_params=pltpu.CompilerParams(needs_layout_passes=False),
        scratch_types=[
            pltpu.VMEM((V,), jnp.float32),
            pltpu.VMEM((BCH,), jnp.int32),
            pltpu.VMEM((BCH,), jnp.float32),
        ],
    )
    def k(ttab_hbm, idx_hbm, out_hbm, vrow, idxb, outb):
        d = lax.axis_index("s") * NC + lax.axis_index("c")

        def per_field(f, _):
            pltpu.sync_copy(ttab_hbm.at[f, d], vrow)

            def per_chunk(h, _):
                pltpu.sync_copy(idx_hbm.at[f, pl.ds(h * BCH, BCH)], idxb)

                def gather_step(i, _):
                    iv = idxb[pl.ds(i * L, L)]
                    outb[pl.ds(i * L, L)] = plsc.load_gather(vrow, [iv])
                    return 0

                lax.fori_loop(0, NVEC, gather_step, 0)
                pltpu.sync_copy(outb,
                                out_hbm.at[f * D + d, pl.ds(h * BCH, BCH)])
                return 0

            lax.fori_loop(0, NBCH, per_chunk, 0)
            return 0

        lax.fori_loop(0, F, per_field, 0)

    return k(ttab, idx_t)


def kernel(indices, tables):
    ttab = jnp.swapaxes(tables, 1, 2)
    idx_t = jnp.swapaxes(indices.astype(jnp.int32), 0, 1)
    out_t = _sc_col_gather(ttab, idx_t)
    return jnp.swapaxes(out_t, 0, 1)


# speedup vs baseline: 4.5058x; 1.3086x over previous
import functools

import jax
import jax.numpy as jnp
from jax import lax
from jax.experimental import pallas as pl
from jax.experimental.pallas import tpu as pltpu
from jax.experimental.pallas import tpu_sc as plsc

F = 26
V = 100000
D = 32
B = 16384

NC, NS, L = 2, 16, 16
NW = NC * NS
BCH = 4096
NBCH = B // BCH
UNROLL = 4


def _sc_col_gather(ttab, idx_t):
    mesh = plsc.VectorSubcoreMesh(core_axis_name="c", subcore_axis_name="s")

    @functools.partial(
        pl.kernel,
        mesh=mesh,
        out_type=jax.ShapeDtypeStruct((F * D, B), jnp.float32),
        compiler_params=pltpu.CompilerParams(needs_layout_passes=False),
        scratch_types=[
            pltpu.VMEM((V,), jnp.float32),
            pltpu.VMEM((BCH,), jnp.int32),
            pltpu.VMEM((BCH,), jnp.int32),
            pltpu.VMEM((BCH,), jnp.float32),
            pltpu.VMEM((BCH,), jnp.float32),
            pltpu.SemaphoreType.DMA,
            pltpu.SemaphoreType.DMA,
            pltpu.SemaphoreType.DMA,
            pltpu.SemaphoreType.DMA,
        ],
    )
    def k(ttab_hbm, idx_hbm, out_hbm, vrow,
          idxb0, idxb1, outb0, outb1, si0, si1, so0, so1):
        d = lax.axis_index("s") * NC + lax.axis_index("c")
        idxb = (idxb0, idxb1)
        outb = (outb0, outb1)
        si = (si0, si1)
        so = (so0, so1)

        def gather_chunk(src, dst):
            def gather_step(i, _):
                for u in range(UNROLL):
                    o = i * L * UNROLL + u * L
                    iv = src[pl.ds(o, L)]
                    dst[pl.ds(o, L)] = plsc.load_gather(vrow, [iv])
                return 0

            lax.fori_loop(0, BCH // (L * UNROLL), gather_step, 0)

        def per_field(f, _):
            row = f * D + d
            pltpu.sync_copy(ttab_hbm.at[f, d], vrow)
            cp_i = [pltpu.async_copy(
                idx_hbm.at[f, pl.ds(0, BCH)], idxb[0], si[0])]
            cp_o = []
            for h in range(NBCH):
                if h + 1 < NBCH:
                    cp_i.append(pltpu.async_copy(
                        idx_hbm.at[f, pl.ds((h + 1) * BCH, BCH)],
                        idxb[(h + 1) % 2], si[(h + 1) % 2]))
                cp_i[h].wait()
                if h >= 2:
                    cp_o[h - 2].wait()
                gather_chunk(idxb[h % 2], outb[h % 2])
                cp_o.append(pltpu.async_copy(
                    outb[h % 2], out_hbm.at[row, pl.ds(h * BCH, BCH)],
                    so[h % 2]))
            cp_o[NBCH - 2].wait()
            cp_o[NBCH - 1].wait()
            return 0

        lax.fori_loop(0, F, per_field, 0)

    return k(ttab, idx_t)


def kernel(indices, tables):
    ttab = jnp.swapaxes(tables, 1, 2)
    idx_t = jnp.swapaxes(indices.astype(jnp.int32), 0, 1)
    out_t = _sc_col_gather(ttab, idx_t)
    return jnp.swapaxes(out_t, 0, 1)


# speedup vs baseline: 4.6580x; 1.0338x over previous
import functools

import jax
import jax.numpy as jnp
from jax import lax
from jax.experimental import pallas as pl
from jax.experimental.pallas import tpu as pltpu
from jax.experimental.pallas import tpu_sc as plsc

F = 26
V = 100000
D = 32
B = 16384

NC, NS, L = 2, 16, 16
NW = NC * NS
BCH = 4096
NBCH = B // BCH
UNROLL = 4


def _sc_col_gather(ttab, idx_t):
    mesh = plsc.VectorSubcoreMesh(core_axis_name="c", subcore_axis_name="s")

    @functools.partial(
        pl.kernel,
        mesh=mesh,
        out_type=jax.ShapeDtypeStruct((F * D, B), jnp.float32),
        compiler_params=pltpu.CompilerParams(needs_layout_passes=False),
        scratch_types=[
            pltpu.VMEM((V,), jnp.float32),
            pltpu.VMEM((BCH,), jnp.int32),
            pltpu.VMEM((BCH,), jnp.int32),
            pltpu.VMEM((BCH,), jnp.float32),
            pltpu.VMEM((BCH,), jnp.float32),
            pltpu.SemaphoreType.DMA,
            pltpu.SemaphoreType.DMA,
            pltpu.SemaphoreType.DMA,
            pltpu.SemaphoreType.DMA,
        ],
    )
    def k(ttab_hbm, idx_hbm, out_hbm, vrow,
          idxb0, idxb1, outb0, outb1, si0, si1, so0, so1):
        d = lax.axis_index("s") * NC + lax.axis_index("c")
        idxb = (idxb0, idxb1)
        outb = (outb0, outb1)
        si = (si0, si1)
        so = (so0, so1)

        def gather_chunk(src, dst):
            def gather_step(i, _):
                for u in range(UNROLL):
                    o = i * L * UNROLL + u * L
                    iv = src[pl.ds(o, L)]
                    dst[pl.ds(o, L)] = plsc.load_gather(vrow, [iv])
                return 0

            lax.fori_loop(0, BCH // (L * UNROLL), gather_step, 0)

        def per_field(f, _):
            row = f * D + d
            cp_i = [pltpu.async_copy(
                idx_hbm.at[f, pl.ds(0, BCH)], idxb[0], si[0]),
                pltpu.async_copy(
                idx_hbm.at[f, pl.ds(BCH, BCH)], idxb[1], si[1])]
            pltpu.sync_copy(ttab_hbm.at[f, d], vrow)
            cp_o = []
            for h in range(NBCH):
                cp_i[h].wait()
                if h >= 2:
                    cp_o[h - 2].wait()
                gather_chunk(idxb[h % 2], outb[h % 2])
                if h + 2 < NBCH:
                    cp_i.append(pltpu.async_copy(
                        idx_hbm.at[f, pl.ds((h + 2) * BCH, BCH)],
                        idxb[h % 2], si[h % 2]))
                cp_o.append(pltpu.async_copy(
                    outb[h % 2], out_hbm.at[row, pl.ds(h * BCH, BCH)],
                    so[h % 2]))
            cp_o[NBCH - 2].wait()
            cp_o[NBCH - 1].wait()
            return 0

        lax.fori_loop(0, F, per_field, 0)

    return k(ttab, idx_t)


def kernel(indices, tables):
    ttab = jnp.swapaxes(tables, 1, 2)
    idx_t = jnp.swapaxes(indices.astype(jnp.int32), 0, 1)
    out_t = _sc_col_gather(ttab, idx_t)
    return jnp.swapaxes(out_t, 0, 1)


# speedup vs baseline: 4.7242x; 1.0142x over previous
import functools

import jax
import jax.numpy as jnp
from jax import lax
from jax.experimental import pallas as pl
from jax.experimental.pallas import tpu as pltpu
from jax.experimental.pallas import tpu_sc as plsc

F = 26
V = 100000
D = 32
B = 16384

NC, NS, L = 2, 16, 16
NW = NC * NS
BCH = 4096
NBCH = B // BCH
UNROLL = 16


def _sc_col_gather(ttab, idx_t):
    mesh = plsc.VectorSubcoreMesh(core_axis_name="c", subcore_axis_name="s")

    @functools.partial(
        pl.kernel,
        mesh=mesh,
        out_type=jax.ShapeDtypeStruct((F * D, B), jnp.float32),
        compiler_params=pltpu.CompilerParams(needs_layout_passes=False),
        scratch_types=[
            pltpu.VMEM((V,), jnp.float32),
            pltpu.VMEM((BCH,), jnp.int32),
            pltpu.VMEM((BCH,), jnp.int32),
            pltpu.VMEM((BCH,), jnp.float32),
            pltpu.VMEM((BCH,), jnp.float32),
            pltpu.SemaphoreType.DMA,
            pltpu.SemaphoreType.DMA,
            pltpu.SemaphoreType.DMA,
            pltpu.SemaphoreType.DMA,
        ],
    )
    def k(ttab_hbm, idx_hbm, out_hbm, vrow,
          idxb0, idxb1, outb0, outb1, si0, si1, so0, so1):
        d = lax.axis_index("s") * NC + lax.axis_index("c")
        idxb = (idxb0, idxb1)
        outb = (outb0, outb1)
        si = (si0, si1)
        so = (so0, so1)

        def gather_chunk(src, dst):
            def gather_step(i, _):
                for u in range(UNROLL):
                    o = i * L * UNROLL + u * L
                    iv = src[pl.ds(o, L)]
                    dst[pl.ds(o, L)] = plsc.load_gather(vrow, [iv])
                return 0

            lax.fori_loop(0, BCH // (L * UNROLL), gather_step, 0)

        def per_field(f, _):
            row = f * D + d
            cp_i = [pltpu.async_copy(
                idx_hbm.at[f, pl.ds(0, BCH)], idxb[0], si[0]),
                pltpu.async_copy(
                idx_hbm.at[f, pl.ds(BCH, BCH)], idxb[1], si[1])]
            pltpu.sync_copy(ttab_hbm.at[f, d], vrow)
            cp_o = []
            for h in range(NBCH):
                cp_i[h].wait()
                if h >= 2:
                    cp_o[h - 2].wait()
                gather_chunk(idxb[h % 2], outb[h % 2])
                if h + 2 < NBCH:
                    cp_i.append(pltpu.async_copy(
                        idx_hbm.at[f, pl.ds((h + 2) * BCH, BCH)],
                        idxb[h % 2], si[h % 2]))
                cp_o.append(pltpu.async_copy(
                    outb[h % 2], out_hbm.at[row, pl.ds(h * BCH, BCH)],
                    so[h % 2]))
            cp_o[NBCH - 2].wait()
            cp_o[NBCH - 1].wait()
            return 0

        lax.fori_loop(0, F, per_field, 0)

    return k(ttab, idx_t)


def kernel(indices, tables):
    ttab = jnp.swapaxes(tables, 1, 2)
    idx_t = jnp.swapaxes(indices.astype(jnp.int32), 0, 1)
    out_t = _sc_col_gather(ttab, idx_t)
    return jnp.swapaxes(out_t, 0, 1)
